# baseline (speedup 1.0000x reference)
import functools

import jax
import jax.numpy as jnp
from jax import lax
from jax.experimental import pallas as pl
from jax.experimental.pallas import tpu as pltpu
from jax.experimental.pallas import tpu_sc as plsc

_TBL_H = 15
_TBL_W = 15
_TBL_PAD = 232
_HW = 4096

_B = 4
_R = 8


def _sc_gather_bias(kflat, tbl_pad):
    info = plsc.get_sparse_core_info()
    nc, ns = info.num_cores, info.num_subcores
    nw = nc * ns
    per_w = _HW // nw
    mesh = plsc.VectorSubcoreMesh(core_axis_name="c", subcore_axis_name="s")

    @functools.partial(
        pl.kernel,
        mesh=mesh,
        compiler_params=pltpu.CompilerParams(needs_layout_passes=False),
        out_type=jax.ShapeDtypeStruct((_HW,), jnp.float32),
        scratch_types=[
            pltpu.VMEM((_TBL_PAD,), jnp.float32),
            pltpu.VMEM((per_w,), jnp.int32),
            pltpu.VMEM((per_w,), jnp.float32),
        ],
    )
    def k(kflat_hbm, tbl_hbm, out_hbm, tbl_v, idx_v, rows_v):
        wid = lax.axis_index("s") * nc + lax.axis_index("c")
        base = wid * per_w
        pltpu.sync_copy(tbl_hbm, tbl_v)
        pltpu.sync_copy(kflat_hbm.at[pl.ds(base, per_w)], idx_v)
        for j in range(per_w // 16):
            idx = idx_v[pl.ds(16 * j, 16)]
            rows_v[pl.ds(16 * j, 16)] = plsc.load_gather(tbl_v, [idx])
        pltpu.sync_copy(rows_v, out_hbm.at[pl.ds(base, per_w)])

    return k(kflat, tbl_pad)


def _stream_kernel(bias_in_ref, x_ref, o_ref, bias_ref, *bufs_and_sems):
    in_bufs = bufs_and_sems[0:_R]
    out_bufs = bufs_and_sems[_R:2 * _R]
    in_sems = bufs_and_sems[2 * _R:3 * _R]
    out_sems = bufs_and_sems[3 * _R:4 * _R]

    bias_ref[...] = bias_in_ref[...]

    nsteps = x_ref.shape[0] // _B
    ngroups = nsteps // _R

    def in_copy(i, slot):
        return pltpu.make_async_copy(
            x_ref.at[pl.ds(i * _B, _B)], in_bufs[slot], in_sems[slot])

    def out_copy(i, slot):
        return pltpu.make_async_copy(
            out_bufs[slot], o_ref.at[pl.ds(i * _B, _B)], out_sems[slot])

    for r in range(_R):
        in_copy(r, r).start()

    def group(g, carry):
        base = g * _R
        for r in range(_R):
            i = base + r
            in_copy(i, r).wait()

            @pl.when(g > 0)
            def _wait_out_slot():
                out_copy(i - _R, r).wait()

            out_bufs[r][...] = in_bufs[r][...] + bias_ref[...][None, None, :, :]
            out_copy(i, r).start()

            @pl.when(g < ngroups - 1)
            def _prefetch():
                in_copy(i + _R, r).start()

        return carry

    lax.fori_loop(0, ngroups, group, 0)

    for r in range(_R):
        out_copy(nsteps - _R + r, r).wait()


def kernel(x, pos_embeddings, relative_indices):
    nb, nh, h, w = x.shape

    kflat = (relative_indices[:, :, 0] * _TBL_W
             + relative_indices[:, :, 1]).reshape(_HW)
    tbl_pad = jnp.pad(pos_embeddings.reshape(-1),
                      (0, _TBL_PAD - _TBL_H * _TBL_W))

    bias = _sc_gather_bias(kflat, tbl_pad).reshape(h, w)

    buf = pltpu.VMEM((_B, nh, h, w), jnp.float32)
    out = pl.pallas_call(
        _stream_kernel,
        in_specs=[
            pl.BlockSpec(memory_space=pltpu.VMEM),
            pl.BlockSpec(memory_space=pl.ANY),
        ],
        out_specs=pl.BlockSpec(memory_space=pl.ANY),
        out_shape=jax.ShapeDtypeStruct(x.shape, jnp.float32),
        scratch_shapes=(
            [pltpu.VMEM((h, w), jnp.float32)]
            + [buf] * (2 * _R)
            + [pltpu.SemaphoreType.DMA] * (2 * _R)
        ),
    )(bias, x)
    return out
